# baseline (speedup 1.0000x reference)
import jax
import jax.numpy as jnp
from jax import lax
from jax.experimental import pallas as pl
from jax.experimental.pallas import tpu as pltpu
from jax.experimental.pallas import tpu_sc as plsc

N = 10000
E = 320000
D = 128
NT = 16
RPT = N // NT
CH = 125
NCH = RPT // CH
EROWS = E // 128
EPAD = 2560


def _rsqrt_newton(x):
    i = plsc.bitcast(x, jnp.int32)
    i = jnp.int32(0x5F3759DF) - lax.shift_right_logical(i, 1)
    y = plsc.bitcast(i, jnp.float32)
    for _ in range(4):
        y = y * (1.5 - 0.5 * x * y * y)
    return y


def _sc_body(x_hbm, rows_hbm, colsc_hbm, out7, a_src, b_prev, zhbm,
             acc, hist, ic0, ic1, ir0, ir1,
             buf0, buf1, ones16, hist_v, diag_sm, sg0, sg1, ss0, ss1):
    c = lax.axis_index("c")
    s = lax.axis_index("s")
    c0s = c == 0
    not_c0s = jnp.logical_not(c0s)
    cN = c * N
    pred16 = (jnp.zeros((16,), jnp.int32) + c) == 0
    i16 = lax.iota(jnp.int32, 16)
    buf_a = buf0
    idxc = (ic0, ic1)
    idxr = (ir0, ir1)
    bufs = (buf0, buf1)
    semg = (sg0, sg1)
    sems = (ss0, ss1)

    def _fill(r, _):
        for jj in range(8):
            buf_a[r, pl.ds(16 * jj, 16)] = jnp.zeros((16,), jnp.float32)
        ones16[r, pl.ds(0, 16)] = jnp.zeros((16,), jnp.float32)
        return _
    lax.fori_loop(0, 128, _fill, 0)
    pltpu.sync_copy(buf_a, zhbm)

    for cc in range(NCH):
        b = s * RPT + cc * CH
        pltpu.sync_copy(buf_a.at[pl.ds(0, CH)], acc.at[pl.ds(b, CH)])
        pltpu.sync_copy(ones16.at[pl.ds(0, CH)], hist.at[pl.ds(b, CH)])

    def _fill1(r, _):
        ones16[r, pl.ds(0, 16)] = jnp.zeros((16,), jnp.float32) + 1.0
        return _
    lax.fori_loop(0, 128, _fill1, 0)
    plsc.subcore_barrier()

    ebase = s * 160

    def _hstart(b, j):
        pltpu.sync_copy(colsc_hbm.at[0, pl.ds(ebase + j, 1)], idxc[b])
        pltpu.async_copy(ones16, hist.at[idxc[b].at[0]], sems[b], add=True)

    def _hwait(b):
        pltpu.make_async_copy(ones16, hist.at[idxc[b].at[0]],
                              sems[b]).wait()

    _hstart(0, 0)
    _hstart(1, 1)

    def _hloop(jo, carry):
        for b in (0, 1):
            j = 2 * jo + b
            _hwait(b)
            _hstart(b, j + 2)
        return carry
    lax.fori_loop(0, 79, _hloop, 0)
    _hwait(0)
    _hwait(1)
    plsc.subcore_barrier()

    def _dq(q, carry):
        pltpu.sync_copy(hist.at[pl.ds(s * RPT + q * 25, 25)], hist_v)

        def _diag(r, carry2):
            h = hist_v[r, pl.ds(0, 16)]
            dm = _rsqrt_newton(h + 1.0)
            di = 1.0 / h
            diag_sm[q * 25 + r] = jnp.max(jnp.where(pred16, dm, di))
            return carry2
        lax.fori_loop(0, 25, _diag, 0)
        return carry
    lax.fori_loop(0, 25, _dq, 0)

    for cc in range(NCH):
        b = s * RPT + cc * CH
        pltpu.sync_copy(x_hbm.at[pl.ds(b, CH)], buf0.at[pl.ds(0, CH)])

        def _scale(r, _, cc=cc):
            dg = diag_sm[cc * CH + r]
            for jj in range(8):
                buf1[r, pl.ds(16 * jj, 16)] = dg * buf0[r, pl.ds(16 * jj, 16)]
            return _
        lax.fori_loop(0, CH, _scale, 0)

        pltpu.async_copy(buf1.at[pl.ds(0, CH)],
                         a_src.at[pl.ds(cN + b, CH)], sg0)

        @pl.when(c0s)
        def _():
            pltpu.async_copy(buf1.at[pl.ds(0, CH)],
                             b_prev.at[pl.ds(cN + b, CH)], ss0)

        @pl.when(not_c0s)
        def _():
            pltpu.async_copy(buf0.at[pl.ds(0, CH)],
                             b_prev.at[pl.ds(cN + b, CH)], ss0)

        pltpu.make_async_copy(buf1.at[pl.ds(0, CH)],
                              a_src.at[pl.ds(cN + b, CH)], sg0).wait()
        pltpu.make_async_copy(buf1.at[pl.ds(0, CH)],
                              b_prev.at[pl.ds(cN + b, CH)], ss0).wait()
    plsc.subcore_barrier()

    def _gstart(b, j):
        pltpu.sync_copy(colsc_hbm.at[c, pl.ds(ebase + j, 1)], idxc[b])
        pltpu.sync_copy(rows_hbm.at[pl.ds(ebase + j, 1)], idxr[b])
        pltpu.async_copy(a_src.at[idxc[b].at[0]], bufs[b], semg[b])

    def _gwait(b):
        pltpu.make_async_copy(a_src.at[idxc[b].at[0]], bufs[b],
                              semg[b]).wait()

    def _sstart(b):
        pltpu.async_copy(bufs[b], acc.at[idxr[b].at[0]], sems[b], add=True)

    def _swait(b):
        pltpu.make_async_copy(bufs[b], acc.at[idxr[b].at[0]],
                              sems[b]).wait()

    for k in range(1, 5):
        _gstart(0, 0)
        _gstart(1, 1)

        def _sloop(jo, carry):
            j = 2 * jo
            _gwait(0)
            _sstart(0)
            _gwait(1)
            _sstart(1)
            _swait(0)
            _gstart(0, j + 2)
            _swait(1)
            _gstart(1, j + 3)
            return carry
        lax.fori_loop(0, 79, _sloop, 0)
        _gwait(0)
        _sstart(0)
        _gwait(1)
        _sstart(1)
        _swait(0)
        _swait(1)
        plsc.subcore_barrier()

        slot = jnp.where(c0s, k - 1, k + 2)
        for cc in range(NCH):
            b = s * RPT + cc * CH
            pltpu.async_copy(acc.at[pl.ds(b, CH)], buf0.at[pl.ds(0, CH)], sg0)
            pltpu.async_copy(b_prev.at[pl.ds(cN + b, CH)],
                             buf1.at[pl.ds(0, CH)], sg1)
            pltpu.make_async_copy(acc.at[pl.ds(b, CH)],
                                  buf0.at[pl.ds(0, CH)], sg0).wait()
            pltpu.make_async_copy(b_prev.at[pl.ds(cN + b, CH)],
                                  buf1.at[pl.ds(0, CH)], sg1).wait()

            def _comb(r, _, cc=cc):
                dg = diag_sm[cc * CH + r]
                coef1 = jnp.where(pred16, dg, 0.5)
                for jj in range(8):
                    o1 = coef1 * (buf0[r, pl.ds(16 * jj, 16)]
                                  + buf1[r, pl.ds(16 * jj, 16)])
                    buf0[r, pl.ds(16 * jj, 16)] = o1
                    buf1[r, pl.ds(16 * jj, 16)] = dg * o1
                return _
            lax.fori_loop(0, CH, _comb, 0)

            if k == 4:
                @pl.when(not_c0s)
                def _():
                    pltpu.async_copy(buf0.at[pl.ds(0, CH)],
                                     out7.at[slot, pl.ds(b, CH)], ss0)
            else:
                pltpu.async_copy(buf0.at[pl.ds(0, CH)],
                                 out7.at[slot, pl.ds(b, CH)], ss0)

            @pl.when(not_c0s)
            def _():
                pltpu.async_copy(buf0.at[pl.ds(0, CH)],
                                 b_prev.at[pl.ds(cN + b, CH)], ss1)

            @pl.when(c0s)
            def _():
                pltpu.async_copy(buf1.at[pl.ds(0, CH)],
                                 b_prev.at[pl.ds(cN + b, CH)], ss1)

            pltpu.async_copy(buf1.at[pl.ds(0, CH)],
                             a_src.at[pl.ds(cN + b, CH)], sg0)
            pltpu.async_copy(zhbm.at[pl.ds(0, CH)], acc.at[pl.ds(b, CH)], sg1)

            if k == 4:
                @pl.when(not_c0s)
                def _():
                    pltpu.make_async_copy(
                        buf0.at[pl.ds(0, CH)],
                        out7.at[slot, pl.ds(b, CH)], ss0).wait()
            else:
                pltpu.make_async_copy(buf0.at[pl.ds(0, CH)],
                                      out7.at[slot, pl.ds(b, CH)], ss0).wait()
            pltpu.make_async_copy(buf0.at[pl.ds(0, CH)],
                                  b_prev.at[pl.ds(cN + b, CH)], ss1).wait()
            pltpu.make_async_copy(buf1.at[pl.ds(0, CH)],
                                  a_src.at[pl.ds(cN + b, CH)], sg0).wait()
            pltpu.make_async_copy(zhbm.at[pl.ds(0, CH)],
                                  acc.at[pl.ds(b, CH)], sg1).wait()
        plsc.subcore_barrier()


def _sc_diffuse(x, rows2d, colsc):
    mesh = plsc.VectorSubcoreMesh(core_axis_name="c", subcore_axis_name="s")
    f = pl.kernel(
        _sc_body,
        out_type=(
            jax.ShapeDtypeStruct((7, N, D), jnp.float32),
            jax.ShapeDtypeStruct((2 * N + 128, D), jnp.float32),
            jax.ShapeDtypeStruct((2 * N, D), jnp.float32),
            jax.ShapeDtypeStruct((128, D), jnp.float32),
        ),
        mesh=mesh,
        compiler_params=pltpu.CompilerParams(
            use_tc_tiling_on_sc=False, needs_layout_passes=False),
        scratch_types=[
            pltpu.VMEM_SHARED((N + 128, D), jnp.float32),
            pltpu.VMEM_SHARED((N + 128, 16), jnp.float32),
            pltpu.VMEM((1, 128), jnp.int32),
            pltpu.VMEM((1, 128), jnp.int32),
            pltpu.VMEM((1, 128), jnp.int32),
            pltpu.VMEM((1, 128), jnp.int32),
            pltpu.VMEM((128, D), jnp.float32),
            pltpu.VMEM((128, D), jnp.float32),
            pltpu.VMEM((128, 16), jnp.float32),
            pltpu.VMEM((25, 16), jnp.float32),
            pltpu.SMEM((RPT,), jnp.float32),
            pltpu.SemaphoreType.DMA,
            pltpu.SemaphoreType.DMA,
            pltpu.SemaphoreType.DMA,
            pltpu.SemaphoreType.DMA,
        ],
    )
    out7, _, _, _ = f(x, rows2d, colsc)
    return out7


def _lrelu(x):
    return jnp.where(x >= 0, x, 0.01 * x)


def _tc_body(mom_ref, x_ref, h_ref, a_ref, w1_ref, b1_ref, w2_ref, b2_ref,
             o_ref):
    x = x_ref[...]
    m = mom_ref[0, 0]
    hs = [_lrelu(h_ref[i]) for i in range(3)]
    p = [h_ref[3 + i] for i in range(4)]
    hs += [jnp.abs(p[i] - p[i + 1]) ** m for i in range(3)]
    a1 = a_ref[0, :D]
    a2 = a_ref[0, D:]
    base = jnp.sum(jnp.maximum(x, 0.0) * a1[None, :], axis=1)
    e = jnp.stack(
        [base + jnp.sum(jnp.maximum(h, 0.0) * a2[None, :], axis=1)
         for h in hs], axis=0)
    mx = jnp.max(e, axis=0, keepdims=True)
    ex = jnp.exp(e - mx)
    att = ex / jnp.sum(ex, axis=0, keepdims=True)
    hp = sum(att[i][:, None] * hs[i] for i in range(6)) * (1.0 / 6.0)
    o = _lrelu(
        lax.dot_general(hp, w1_ref[...], (((1,), (1,)), ((), ())),
                        preferred_element_type=jnp.float32) + b1_ref[0][None, :])
    o = _lrelu(
        lax.dot_general(o, w2_ref[...], (((1,), (1,)), ((), ())),
                        preferred_element_type=jnp.float32) + b2_ref[0][None, :])
    o_ref[...] = o


def _tc_epilogue(x, h7, a_r, w1, b1r, w2, b2r, mom):
    bn = 1000
    grid = (N // bn,)
    return pl.pallas_call(
        _tc_body,
        grid=grid,
        in_specs=[
            pl.BlockSpec(memory_space=pltpu.SMEM),
            pl.BlockSpec((bn, D), lambda i: (i, 0)),
            pl.BlockSpec((7, bn, D), lambda i: (0, i, 0)),
            pl.BlockSpec((1, 2 * D), lambda i: (0, 0)),
            pl.BlockSpec((D, D), lambda i: (0, 0)),
            pl.BlockSpec((1, D), lambda i: (0, 0)),
            pl.BlockSpec((D, D), lambda i: (0, 0)),
            pl.BlockSpec((1, D), lambda i: (0, 0)),
        ],
        out_specs=pl.BlockSpec((bn, D), lambda i: (i, 0)),
        out_shape=jax.ShapeDtypeStruct((N, D), jnp.float32),
    )(mom, x, h7, a_r, w1, b1r, w2, b2r)


def kernel(X, edge_index, a, W1, b1, W2, b2, moment):
    padv = jnp.broadcast_to(
        jnp.arange(128, dtype=jnp.int32)[None, :] + N,
        (EPAD - EROWS, 128))
    rows2d = jnp.concatenate([edge_index[0].reshape(EROWS, 128), padv], 0)
    cols2d = jnp.concatenate([edge_index[1].reshape(EROWS, 128), padv], 0)
    colsc = jnp.stack([cols2d, cols2d + N], axis=0)
    h7 = _sc_diffuse(X, rows2d, colsc)
    mom = jnp.asarray(moment, jnp.float32).reshape(1, 1)
    a_r = a.reshape(1, 2 * D)
    return _tc_epilogue(X, h7, a_r, W1, b1.reshape(1, D), W2, b2.reshape(1, D),
                        mom)


# speedup vs baseline: 1.0336x; 1.0336x over previous
import jax
import jax.numpy as jnp
from jax import lax
from jax.experimental import pallas as pl
from jax.experimental.pallas import tpu as pltpu
from jax.experimental.pallas import tpu_sc as plsc

N = 10000
E = 320000
D = 128
NT = 16
RPT = N // NT
CH = 125
NCH = RPT // CH
EROWS = E // 128
EPAD = 2560


def _rsqrt_newton(x):
    i = plsc.bitcast(x, jnp.int32)
    i = jnp.int32(0x5F3759DF) - lax.shift_right_logical(i, 1)
    y = plsc.bitcast(i, jnp.float32)
    for _ in range(4):
        y = y * (1.5 - 0.5 * x * y * y)
    return y


def _sc_body(x_hbm, rows_hbm, colsc_hbm, out7, a_src, b_prev, zhbm,
             acc, hist, ic0, ic1, ir0, ir1,
             buf0, buf1, ones16, hist_v, diag_sm, sg0, sg1, ss0, ss1):
    c = lax.axis_index("c")
    s = lax.axis_index("s")
    c0s = c == 0
    not_c0s = jnp.logical_not(c0s)
    cN = c * N
    pred16 = (jnp.zeros((16,), jnp.int32) + c) == 0
    i16 = lax.iota(jnp.int32, 16)
    buf_a = buf0
    idxc = (ic0, ic1)
    idxr = (ir0, ir1)
    bufs = (buf0, buf1)
    semg = (sg0, sg1)
    sems = (ss0, ss1)

    def _fill(r, _):
        for jj in range(8):
            buf_a[r, pl.ds(16 * jj, 16)] = jnp.zeros((16,), jnp.float32)
        ones16[r, pl.ds(0, 16)] = jnp.zeros((16,), jnp.float32)
        return _
    lax.fori_loop(0, 128, _fill, 0)
    pltpu.sync_copy(buf_a, zhbm)

    for cc in range(NCH):
        b = s * RPT + cc * CH
        pltpu.sync_copy(buf_a.at[pl.ds(0, CH)], acc.at[pl.ds(b, CH)])
        pltpu.sync_copy(ones16.at[pl.ds(0, CH)], hist.at[pl.ds(b, CH)])

    def _fill1(r, _):
        ones16[r, pl.ds(0, 16)] = jnp.zeros((16,), jnp.float32) + 1.0
        return _
    lax.fori_loop(0, 128, _fill1, 0)
    plsc.subcore_barrier()

    ebase = s * 156 + jnp.minimum(s, 4)
    ecnt = jnp.where(s < 4, 157, 156)

    def _hstart(b, j):
        pltpu.sync_copy(colsc_hbm.at[0, pl.ds(ebase + j, 1)], idxc[b])
        pltpu.async_copy(ones16, hist.at[idxc[b].at[0]], sems[b], add=True)

    def _hwait(b):
        pltpu.make_async_copy(ones16, hist.at[idxc[b].at[0]],
                              sems[b]).wait()

    _hstart(0, 0)
    _hstart(1, 1)

    def _hloop(jo, carry):
        for b in (0, 1):
            j = 2 * jo + b

            @pl.when(j + 2 < ecnt)
            def _(b=b, j=j):
                _hwait(b)
                _hstart(b, j + 2)
        return carry
    lax.fori_loop(0, 79, _hloop, 0)
    _hwait(0)
    _hwait(1)
    plsc.subcore_barrier()

    def _dq(q, carry):
        pltpu.sync_copy(hist.at[pl.ds(s * RPT + q * 25, 25)], hist_v)

        def _diag(r, carry2):
            h = hist_v[r, pl.ds(0, 16)]
            dm = _rsqrt_newton(h + 1.0)
            di = 1.0 / h
            diag_sm[q * 25 + r] = jnp.max(jnp.where(pred16, dm, di))
            return carry2
        lax.fori_loop(0, 25, _diag, 0)
        return carry
    lax.fori_loop(0, 25, _dq, 0)

    for cc in range(NCH):
        b = s * RPT + cc * CH
        pltpu.sync_copy(x_hbm.at[pl.ds(b, CH)], buf0.at[pl.ds(0, CH)])

        def _scale(r, _, cc=cc):
            dg = diag_sm[cc * CH + r]
            for jj in range(8):
                buf1[r, pl.ds(16 * jj, 16)] = dg * buf0[r, pl.ds(16 * jj, 16)]
            return _
        lax.fori_loop(0, CH, _scale, 0)

        pltpu.async_copy(buf1.at[pl.ds(0, CH)],
                         a_src.at[pl.ds(cN + b, CH)], sg0)

        @pl.when(c0s)
        def _():
            pltpu.async_copy(buf1.at[pl.ds(0, CH)],
                             b_prev.at[pl.ds(cN + b, CH)], ss0)

        @pl.when(not_c0s)
        def _():
            pltpu.async_copy(buf0.at[pl.ds(0, CH)],
                             b_prev.at[pl.ds(cN + b, CH)], ss0)

        pltpu.make_async_copy(buf1.at[pl.ds(0, CH)],
                              a_src.at[pl.ds(cN + b, CH)], sg0).wait()
        pltpu.make_async_copy(buf1.at[pl.ds(0, CH)],
                              b_prev.at[pl.ds(cN + b, CH)], ss0).wait()
    plsc.subcore_barrier()

    def _gstart(b, j):
        pltpu.sync_copy(colsc_hbm.at[c, pl.ds(ebase + j, 1)], idxc[b])
        pltpu.sync_copy(rows_hbm.at[pl.ds(ebase + j, 1)], idxr[b])
        pltpu.async_copy(a_src.at[idxc[b].at[0]], bufs[b], semg[b])

    def _gwait(b):
        pltpu.make_async_copy(a_src.at[idxc[b].at[0]], bufs[b],
                              semg[b]).wait()

    def _sstart(b):
        pltpu.async_copy(bufs[b], acc.at[idxr[b].at[0]], sems[b], add=True)

    def _swait(b):
        pltpu.make_async_copy(bufs[b], acc.at[idxr[b].at[0]],
                              sems[b]).wait()

    for k in range(1, 5):
        _gstart(0, 0)
        _gstart(1, 1)

        def _sloop(jo, carry):
            for b in (0, 1):
                j = 2 * jo + b

                @pl.when(j < ecnt)
                def _(b=b):
                    _gwait(b)
                    _sstart(b)

                @pl.when(j + 2 < ecnt)
                def _(b=b, j=j):
                    _swait(b)
                    _gstart(b, j + 2)
            return carry
        lax.fori_loop(0, 79, _sloop, 0)
        _swait(0)
        _swait(1)
        plsc.subcore_barrier()

        slot = jnp.where(c0s, k - 1, k + 2)
        for cc in range(NCH):
            b = s * RPT + cc * CH
            pltpu.async_copy(acc.at[pl.ds(b, CH)], buf0.at[pl.ds(0, CH)], sg0)
            pltpu.async_copy(b_prev.at[pl.ds(cN + b, CH)],
                             buf1.at[pl.ds(0, CH)], sg1)
            pltpu.make_async_copy(acc.at[pl.ds(b, CH)],
                                  buf0.at[pl.ds(0, CH)], sg0).wait()
            pltpu.make_async_copy(b_prev.at[pl.ds(cN + b, CH)],
                                  buf1.at[pl.ds(0, CH)], sg1).wait()

            def _comb(r, _, cc=cc):
                dg = diag_sm[cc * CH + r]
                coef1 = jnp.where(pred16, dg, 0.5)
                for jj in range(8):
                    o1 = coef1 * (buf0[r, pl.ds(16 * jj, 16)]
                                  + buf1[r, pl.ds(16 * jj, 16)])
                    buf0[r, pl.ds(16 * jj, 16)] = o1
                    buf1[r, pl.ds(16 * jj, 16)] = dg * o1
                return _
            lax.fori_loop(0, CH, _comb, 0)

            if k == 4:
                @pl.when(not_c0s)
                def _():
                    pltpu.async_copy(buf0.at[pl.ds(0, CH)],
                                     out7.at[slot, pl.ds(b, CH)], ss0)
            else:
                pltpu.async_copy(buf0.at[pl.ds(0, CH)],
                                 out7.at[slot, pl.ds(b, CH)], ss0)

            @pl.when(not_c0s)
            def _():
                pltpu.async_copy(buf0.at[pl.ds(0, CH)],
                                 b_prev.at[pl.ds(cN + b, CH)], ss1)

            @pl.when(c0s)
            def _():
                pltpu.async_copy(buf1.at[pl.ds(0, CH)],
                                 b_prev.at[pl.ds(cN + b, CH)], ss1)

            pltpu.async_copy(buf1.at[pl.ds(0, CH)],
                             a_src.at[pl.ds(cN + b, CH)], sg0)
            pltpu.async_copy(zhbm.at[pl.ds(0, CH)], acc.at[pl.ds(b, CH)], sg1)

            if k == 4:
                @pl.when(not_c0s)
                def _():
                    pltpu.make_async_copy(
                        buf0.at[pl.ds(0, CH)],
                        out7.at[slot, pl.ds(b, CH)], ss0).wait()
            else:
                pltpu.make_async_copy(buf0.at[pl.ds(0, CH)],
                                      out7.at[slot, pl.ds(b, CH)], ss0).wait()
            pltpu.make_async_copy(buf0.at[pl.ds(0, CH)],
                                  b_prev.at[pl.ds(cN + b, CH)], ss1).wait()
            pltpu.make_async_copy(buf1.at[pl.ds(0, CH)],
                                  a_src.at[pl.ds(cN + b, CH)], sg0).wait()
            pltpu.make_async_copy(zhbm.at[pl.ds(0, CH)],
                                  acc.at[pl.ds(b, CH)], sg1).wait()
        plsc.subcore_barrier()


def _sc_diffuse(x, rows2d, colsc):
    mesh = plsc.VectorSubcoreMesh(core_axis_name="c", subcore_axis_name="s")
    f = pl.kernel(
        _sc_body,
        out_type=(
            jax.ShapeDtypeStruct((7, N, D), jnp.float32),
            jax.ShapeDtypeStruct((2 * N, D), jnp.float32),
            jax.ShapeDtypeStruct((2 * N, D), jnp.float32),
            jax.ShapeDtypeStruct((128, D), jnp.float32),
        ),
        mesh=mesh,
        compiler_params=pltpu.CompilerParams(
            use_tc_tiling_on_sc=False, needs_layout_passes=False),
        scratch_types=[
            pltpu.VMEM_SHARED((N, D), jnp.float32),
            pltpu.VMEM_SHARED((N, 16), jnp.float32),
            pltpu.VMEM((1, 128), jnp.int32),
            pltpu.VMEM((1, 128), jnp.int32),
            pltpu.VMEM((1, 128), jnp.int32),
            pltpu.VMEM((1, 128), jnp.int32),
            pltpu.VMEM((128, D), jnp.float32),
            pltpu.VMEM((128, D), jnp.float32),
            pltpu.VMEM((128, 16), jnp.float32),
            pltpu.VMEM((25, 16), jnp.float32),
            pltpu.SMEM((RPT,), jnp.float32),
            pltpu.SemaphoreType.DMA,
            pltpu.SemaphoreType.DMA,
            pltpu.SemaphoreType.DMA,
            pltpu.SemaphoreType.DMA,
        ],
    )
    out7, _, _, _ = f(x, rows2d, colsc)
    return out7


def _lrelu(x):
    return jnp.where(x >= 0, x, 0.01 * x)


def _tc_body(mom_ref, x_ref, h_ref, a_ref, w1_ref, b1_ref, w2_ref, b2_ref,
             o_ref):
    x = x_ref[...]
    m = mom_ref[0, 0]
    hs = [_lrelu(h_ref[i]) for i in range(3)]
    p = [h_ref[3 + i] for i in range(4)]
    hs += [jnp.abs(p[i] - p[i + 1]) ** m for i in range(3)]
    a1 = a_ref[0, :D]
    a2 = a_ref[0, D:]
    base = jnp.sum(jnp.maximum(x, 0.0) * a1[None, :], axis=1)
    e = jnp.stack(
        [base + jnp.sum(jnp.maximum(h, 0.0) * a2[None, :], axis=1)
         for h in hs], axis=0)
    mx = jnp.max(e, axis=0, keepdims=True)
    ex = jnp.exp(e - mx)
    att = ex / jnp.sum(ex, axis=0, keepdims=True)
    hp = sum(att[i][:, None] * hs[i] for i in range(6)) * (1.0 / 6.0)
    o = _lrelu(
        lax.dot_general(hp, w1_ref[...], (((1,), (1,)), ((), ())),
                        preferred_element_type=jnp.float32) + b1_ref[0][None, :])
    o = _lrelu(
        lax.dot_general(o, w2_ref[...], (((1,), (1,)), ((), ())),
                        preferred_element_type=jnp.float32) + b2_ref[0][None, :])
    o_ref[...] = o


def _tc_epilogue(x, h7, a_r, w1, b1r, w2, b2r, mom):
    bn = 1000
    grid = (N // bn,)
    return pl.pallas_call(
        _tc_body,
        grid=grid,
        in_specs=[
            pl.BlockSpec(memory_space=pltpu.SMEM),
            pl.BlockSpec((bn, D), lambda i: (i, 0)),
            pl.BlockSpec((7, bn, D), lambda i: (0, i, 0)),
            pl.BlockSpec((1, 2 * D), lambda i: (0, 0)),
            pl.BlockSpec((D, D), lambda i: (0, 0)),
            pl.BlockSpec((1, D), lambda i: (0, 0)),
            pl.BlockSpec((D, D), lambda i: (0, 0)),
            pl.BlockSpec((1, D), lambda i: (0, 0)),
        ],
        out_specs=pl.BlockSpec((bn, D), lambda i: (i, 0)),
        out_shape=jax.ShapeDtypeStruct((N, D), jnp.float32),
    )(mom, x, h7, a_r, w1, b1r, w2, b2r)


def kernel(X, edge_index, a, W1, b1, W2, b2, moment):
    rows2d = edge_index[0].reshape(EROWS, 128)
    cols2d = edge_index[1].reshape(EROWS, 128)
    colsc = jnp.stack([cols2d, cols2d + N], axis=0)
    h7 = _sc_diffuse(X, rows2d, colsc)
    mom = jnp.asarray(moment, jnp.float32).reshape(1, 1)
    a_r = a.reshape(1, 2 * D)
    return _tc_epilogue(X, h7, a_r, W1, b1.reshape(1, D), W2, b2.reshape(1, D),
                        mom)
